# Initial kernel scaffold; baseline (speedup 1.0000x reference)
import jax
import jax.numpy as jnp
from jax.experimental import pallas as pl


def kernel(x, table, pos):
    raise NotImplementedError("write your pallas kernel here")


# speedup vs baseline: 1.9825x; 1.9825x over previous
import functools

import jax
import jax.numpy as jnp
from jax import lax
from jax.experimental import pallas as pl
from jax.experimental.pallas import tpu as pltpu
from jax.experimental.pallas import tpu_sc as plsc

B = 16384
L = 26
E = 32
NW = 32
BPW = B // NW
CHUNK = 64
NCHUNK = BPW // CHUNK
ROWS = CHUNK * L
NSTREAM = ROWS // 128
IDX_ROWS = BPW * L // 128


def _tree_sum(terms):
    while len(terms) > 1:
        nxt = [terms[i] + terms[i + 1] for i in range(0, len(terms) - 1, 2)]
        if len(terms) % 2:
            nxt.append(terms[-1])
        terms = nxt
    return terms[0]


def _body(x_hbm, table_hbm, pos_hbm, out_hbm,
          idx_v, rows_v0, rows_v1, out_v0, out_v1, pos_v,
          sem_g0, sem_g1, sem_o0, sem_o1):
    c = lax.axis_index("c")
    s = lax.axis_index("s")
    wid = c * 16 + s
    base_b = wid * BPW

    pltpu.sync_copy(x_hbm.at[wid], idx_v)
    pltpu.sync_copy(pos_hbm, pos_v)
    pos0 = pos_v[0]
    pos1 = pos_v[1]
    inv = jnp.float32(1.0 / L)

    def fire(g, rows_v, sem):
        return [
            pltpu.async_copy(
                table_hbm.at[idx_v.at[g * NSTREAM + j]],
                rows_v.at[pl.ds(j * 128, 128)],
                sem,
            )
            for j in range(NSTREAM)
        ]

    def reduce_chunk(rows_v, out_v):
        def body_b(b, carry):
            r0 = b * L
            t0 = [rows_v[r0 + l, pl.ds(0, 16)] for l in range(L)]
            t1 = [rows_v[r0 + l, pl.ds(16, 16)] for l in range(L)]
            out_v[b, pl.ds(0, 16)] = _tree_sum(t0) * inv + pos0
            out_v[b, pl.ds(16, 16)] = _tree_sum(t1) * inv + pos1
            return carry
        lax.fori_loop(0, CHUNK, body_b, 0)

    bufs = [(rows_v0, sem_g0, out_v0, sem_o0),
            (rows_v1, sem_g1, out_v1, sem_o1)]
    gather_hs = {0: fire(0, rows_v0, sem_g0)}
    out_hs = {}
    for g in range(NCHUNK):
        rows_v, _, out_v, sem_o = bufs[g % 2]
        if g + 1 < NCHUNK:
            nrows, nsem, _, _ = bufs[(g + 1) % 2]
            gather_hs[g + 1] = fire(g + 1, nrows, nsem)
        for h in gather_hs[g]:
            h.wait()
        if g >= 2:
            out_hs[g - 2].wait()
        reduce_chunk(rows_v, out_v)
        out_hs[g] = pltpu.async_copy(
            out_v, out_hbm.at[pl.ds(base_b + g * CHUNK, CHUNK)], sem_o)
    out_hs[NCHUNK - 2].wait()
    out_hs[NCHUNK - 1].wait()


@jax.jit
def kernel(x, table, pos):
    x2 = x.astype(jnp.int32).reshape(NW, IDX_ROWS, 128)
    pos2 = pos.reshape(2, 16)
    mesh = plsc.VectorSubcoreMesh(core_axis_name="c", subcore_axis_name="s")
    k = pl.kernel(
        _body,
        out_type=jax.ShapeDtypeStruct((B, E), jnp.float32),
        mesh=mesh,
        scratch_types=[
            pltpu.VMEM((IDX_ROWS, 128), jnp.int32),
            pltpu.VMEM((ROWS, E), jnp.float32),
            pltpu.VMEM((ROWS, E), jnp.float32),
            pltpu.VMEM((CHUNK, E), jnp.float32),
            pltpu.VMEM((CHUNK, E), jnp.float32),
            pltpu.VMEM((2, 16), jnp.float32),
            pltpu.SemaphoreType.DMA,
            pltpu.SemaphoreType.DMA,
            pltpu.SemaphoreType.DMA,
            pltpu.SemaphoreType.DMA,
        ],
        compiler_params=pltpu.CompilerParams(use_tc_tiling_on_sc=False),
    )
    return k(x2, table, pos2)


# speedup vs baseline: 7.5554x; 3.8110x over previous
import jax
import jax.numpy as jnp
from jax import lax
from jax.experimental import pallas as pl
from jax.experimental.pallas import tpu as pltpu
from jax.experimental.pallas import tpu_sc as plsc

B = 16384
L = 26
E = 32
NW = 32
BPW = B // NW
PACK_B = 8192
PACK_NM = 31
PACK_ROWS = PACK_B * PACK_NM
CHUNK = 64
NCHUNK = BPW // CHUNK
ROWS = CHUNK * L


def _pack_body(i_ref, o_ref):
    x = i_ref[...]
    xq = jnp.concatenate(
        [x[:, u * PACK_B:(u + 1) * PACK_B] for u in range(4)], axis=0)
    o_ref[...] = xq.T


def _transpose_pack(tt):
    return pl.pallas_call(
        _pack_body,
        grid=(PACK_NM,),
        in_specs=[pl.BlockSpec((32, 4 * PACK_B), lambda m: (0, m))],
        out_specs=pl.BlockSpec((PACK_B, 128), lambda m: (m, 0)),
        out_shape=jax.ShapeDtypeStruct((PACK_ROWS, 128), jnp.float32),
    )(tt)


def _tree_sum(terms):
    while len(terms) > 1:
        nxt = [terms[i] + terms[i + 1] for i in range(0, len(terms) - 1, 2)]
        if len(terms) % 2:
            nxt.append(terms[-1])
        terms = nxt
    return terms[0]


def _body(x_hbm, table_hbm, pos_hbm, out_hbm,
          idx_v, rows_v0, rows_v1, out_v0, out_v1, pos_v,
          sem_g0, sem_g1, sem_o0, sem_o1):
    c = lax.axis_index("c")
    s = lax.axis_index("s")
    wid = c * 16 + s
    base_b = wid * BPW

    pltpu.sync_copy(x_hbm.at[pl.ds(0, L), pl.ds(wid * BPW, BPW)], idx_v)
    pltpu.sync_copy(pos_hbm, pos_v)
    pos0 = pos_v[0]
    pos1 = pos_v[1]
    inv = jnp.float32(1.0 / L)

    def permute_rows(g):
        def row_fn(l, carry):
            for r in range(CHUNK // 16):
                sl = pl.ds(g * CHUNK + 16 * r, 16)
                v = idx_v[l, sl]
                idx_v[l, sl] = ((v >> 15) << 15) + ((v & 8191) << 2) \
                    + ((v >> 13) & 3)
            return carry
        lax.fori_loop(0, L, row_fn, 0)

    def fire(g, rows_v, sem):
        return [
            pltpu.async_copy(
                table_hbm.at[idx_v.at[l, pl.ds(g * CHUNK, CHUNK)]],
                rows_v.at[pl.ds(l * CHUNK, CHUNK)],
                sem,
            )
            for l in range(L)
        ]

    def reduce_chunk(rows_v, out_v):
        def body_b(b, carry):
            t0 = [rows_v[l * CHUNK + b, pl.ds(0, 16)] for l in range(L)]
            t1 = [rows_v[l * CHUNK + b, pl.ds(16, 16)] for l in range(L)]
            out_v[b, pl.ds(0, 16)] = _tree_sum(t0) * inv + pos0
            out_v[b, pl.ds(16, 16)] = _tree_sum(t1) * inv + pos1
            return carry
        lax.fori_loop(0, CHUNK, body_b, 0)

    bufs = [(rows_v0, sem_g0, out_v0, sem_o0),
            (rows_v1, sem_g1, out_v1, sem_o1)]
    permute_rows(0)
    gather_hs = {0: fire(0, rows_v0, sem_g0)}
    out_hs = {}
    for g in range(NCHUNK):
        rows_v, _, out_v, sem_o = bufs[g % 2]
        if g + 1 < NCHUNK:
            nrows, nsem, _, _ = bufs[(g + 1) % 2]
            permute_rows(g + 1)
            gather_hs[g + 1] = fire(g + 1, nrows, nsem)
        for h in gather_hs[g]:
            h.wait()
        if g >= 2:
            out_hs[g - 2].wait()
        reduce_chunk(rows_v, out_v)
        out_hs[g] = pltpu.async_copy(
            out_v, out_hbm.at[pl.ds(base_b + g * CHUNK, CHUNK)], sem_o)
    out_hs[NCHUNK - 2].wait()
    out_hs[NCHUNK - 1].wait()


@jax.jit
def kernel(x, table, pos):
    xt = x.astype(jnp.int32).T
    x2 = jnp.concatenate([xt, jnp.zeros((32 - L, B), jnp.int32)], axis=0)
    pos2 = pos.reshape(2, 16)
    t_lin = _transpose_pack(table.T).reshape(4 * PACK_ROWS, E)
    mesh = plsc.VectorSubcoreMesh(core_axis_name="c", subcore_axis_name="s")
    k = pl.kernel(
        _body,
        out_type=jax.ShapeDtypeStruct((B, E), jnp.float32),
        mesh=mesh,
        scratch_types=[
            pltpu.VMEM((L, BPW), jnp.int32),
            pltpu.VMEM((ROWS, E), jnp.float32),
            pltpu.VMEM((ROWS, E), jnp.float32),
            pltpu.VMEM((CHUNK, E), jnp.float32),
            pltpu.VMEM((CHUNK, E), jnp.float32),
            pltpu.VMEM((2, 16), jnp.float32),
            pltpu.SemaphoreType.DMA,
            pltpu.SemaphoreType.DMA,
            pltpu.SemaphoreType.DMA,
            pltpu.SemaphoreType.DMA,
        ],
        compiler_params=pltpu.CompilerParams(use_tc_tiling_on_sc=False),
    )
    return k(x2, t_lin, pos2)
